# Initial kernel scaffold; baseline (speedup 1.0000x reference)
import jax
import jax.numpy as jnp
from jax.experimental import pallas as pl


def kernel(x, WL0, bL0, WG0, bG0, lin0W, lin0b, WL1, bL1, WG1, bG1, lin1W, lin1b, gamma, beta, finW, finb, G_edge_index, L_edge_index):
    raise NotImplementedError("write your pallas kernel here")


# speedup vs baseline: 14.8945x; 14.8945x over previous
import functools

import jax
import jax.numpy as jnp
from jax import lax
from jax.experimental import pallas as pl
from jax.experimental.pallas import tpu as pltpu
from jax.experimental.pallas import tpu_sc as plsc

_N = 10000
_E = 320000
_H = 128
_NP = 10240
_NS = 16
_RPT = _NP // _NS
_EPT = _E // _NS
_CH = 80
_CHUNKS = _EPT // _CH


def _sc_degree(dstL, dstG):
    mesh = plsc.VectorSubcoreMesh(core_axis_name="c", subcore_axis_name="s")

    @functools.partial(
        pl.kernel,
        out_type=(jax.ShapeDtypeStruct((_NP,), jnp.float32),
                  jax.ShapeDtypeStruct((_NP,), jnp.float32)),
        mesh=mesh,
        scratch_types=[
            pltpu.VMEM((_CH,), jnp.int32),
            pltpu.VMEM((_CH,), jnp.int32),
            pltpu.VMEM((_CH,), jnp.float32),
            pltpu.VMEM((_RPT,), jnp.float32),
            pltpu.VMEM_SHARED((_NP,), jnp.float32),
            pltpu.SemaphoreType.DMA,
            pltpu.SemaphoreType.DMA,
        ],
    )
    def deg_kernel(dstL_hbm, dstG_hbm, outL_hbm, outG_hbm,
                   idxA, idxB, ones_v, stage_v, deg_sh, semA, semB):
        c = lax.axis_index("c")
        s = lax.axis_index("s")

        def run(dst_hbm, out_hbm):
            base = s * _EPT

            def fill(j, carry):
                ones_v[pl.ds(j * 16, 16)] = jnp.ones((16,), jnp.float32)
                stage_v[pl.ds(j * 16, 16)] = jnp.zeros((16,), jnp.float32)
                return carry

            lax.fori_loop(0, _CH // 16, fill, 0)

            def zfill(j, carry):
                stage_v[pl.ds(j * 16, 16)] = jnp.zeros((16,), jnp.float32)
                return carry

            lax.fori_loop(_CH // 16, _RPT // 16, zfill, 0)
            pltpu.sync_copy(stage_v, deg_sh.at[pl.ds(s * _RPT, _RPT)])
            plsc.subcore_barrier()

            pltpu.async_copy(dst_hbm.at[pl.ds(base, _CH)], idxA, semA)

            def body(i, carry):
                g0 = 2 * i
                pltpu.make_async_copy(
                    dst_hbm.at[pl.ds(base, _CH)], idxA, semA).wait()
                pltpu.async_copy(
                    dst_hbm.at[pl.ds(base + (g0 + 1) * _CH, _CH)], idxB, semB)
                pltpu.sync_copy(ones_v, deg_sh.at[idxA], add=True)
                pltpu.make_async_copy(
                    dst_hbm.at[pl.ds(base, _CH)], idxB, semB).wait()

                @pl.when(g0 + 2 < _CHUNKS)
                def _():
                    pltpu.async_copy(
                        dst_hbm.at[pl.ds(base + (g0 + 2) * _CH, _CH)],
                        idxA, semA)

                pltpu.sync_copy(ones_v, deg_sh.at[idxB], add=True)
                return carry

            lax.fori_loop(0, _CHUNKS // 2, body, 0)
            plsc.subcore_barrier()
            pltpu.sync_copy(deg_sh.at[pl.ds(s * _RPT, _RPT)],
                            out_hbm.at[pl.ds(s * _RPT, _RPT)])

        @pl.when(c == 0)
        def _():
            run(dstL_hbm, outL_hbm)

        @pl.when(c == 1)
        def _():
            run(dstG_hbm, outG_hbm)

    return deg_kernel(dstL, dstG)


def _sc_gather_scatter(hL, hG, sLi, dLi, sGi, dGi):
    mesh = plsc.VectorSubcoreMesh(core_axis_name="c", subcore_axis_name="s")

    @functools.partial(
        pl.kernel,
        out_type=(jax.ShapeDtypeStruct((_NP, _H), jnp.float32),
                  jax.ShapeDtypeStruct((_NP, _H), jnp.float32)),
        mesh=mesh,
        scratch_types=[
            pltpu.VMEM((_CH,), jnp.int32),
            pltpu.VMEM((_CH,), jnp.int32),
            pltpu.VMEM((_CH,), jnp.int32),
            pltpu.VMEM((_CH,), jnp.int32),
            pltpu.VMEM((_CH, _H), jnp.float32),
            pltpu.VMEM((_CH, _H), jnp.float32),
            pltpu.VMEM_SHARED((_NP, _H), jnp.float32),
            pltpu.SemaphoreType.DMA,
            pltpu.SemaphoreType.DMA,
            pltpu.SemaphoreType.DMA,
            pltpu.SemaphoreType.DMA,
            pltpu.SemaphoreType.DMA,
            pltpu.SemaphoreType.DMA,
        ],
    )
    def scat_kernel(hL_hbm, hG_hbm, sL_hbm, dL_hbm, sG_hbm, dG_hbm,
                    outL_hbm, outG_hbm,
                    srcA, srcB, dstA, dstB, rowsA, rowsB, acc_sh,
                    semSA, semSB, semDA, semDB, semGA, semGB):
        c = lax.axis_index("c")
        s = lax.axis_index("s")

        def run(h_hbm, se_hbm, de_hbm, out_hbm):
            base = s * _EPT

            def zrow(j, carry):
                rowsA[j // 8, pl.ds((j % 8) * 16, 16)] = (
                    jnp.zeros((16,), jnp.float32))
                return carry

            lax.fori_loop(0, _CH * 8, zrow, 0)

            def zdma(j, carry):
                pltpu.sync_copy(rowsA,
                                acc_sh.at[pl.ds(s * _RPT + j * _CH, _CH)])
                return carry

            lax.fori_loop(0, _RPT // _CH, zdma, 0)
            plsc.subcore_barrier()

            pltpu.async_copy(se_hbm.at[pl.ds(base, _CH)], srcA, semSA)
            pltpu.async_copy(de_hbm.at[pl.ds(base, _CH)], dstA, semDA)

            def body(i, carry):
                g0 = 2 * i
                offB = base + (g0 + 1) * _CH
                offA2 = base + (g0 + 2) * _CH
                pltpu.make_async_copy(
                    se_hbm.at[pl.ds(base, _CH)], srcA, semSA).wait()
                ga = pltpu.async_copy(h_hbm.at[srcA], rowsA, semGA)
                pltpu.async_copy(se_hbm.at[pl.ds(offB, _CH)], srcB, semSB)
                pltpu.async_copy(de_hbm.at[pl.ds(offB, _CH)], dstB, semDB)
                ga.wait()
                pltpu.make_async_copy(
                    de_hbm.at[pl.ds(base, _CH)], dstA, semDA).wait()
                pltpu.sync_copy(rowsA, acc_sh.at[dstA], add=True)
                pltpu.make_async_copy(
                    se_hbm.at[pl.ds(base, _CH)], srcB, semSB).wait()
                gb = pltpu.async_copy(h_hbm.at[srcB], rowsB, semGB)

                @pl.when(g0 + 2 < _CHUNKS)
                def _():
                    pltpu.async_copy(se_hbm.at[pl.ds(offA2, _CH)],
                                     srcA, semSA)
                    pltpu.async_copy(de_hbm.at[pl.ds(offA2, _CH)],
                                     dstA, semDA)

                gb.wait()
                pltpu.make_async_copy(
                    de_hbm.at[pl.ds(base, _CH)], dstB, semDB).wait()
                pltpu.sync_copy(rowsB, acc_sh.at[dstB], add=True)
                return carry

            lax.fori_loop(0, _CHUNKS // 2, body, 0)
            plsc.subcore_barrier()
            pltpu.sync_copy(acc_sh.at[pl.ds(s * _RPT, _RPT)],
                            out_hbm.at[pl.ds(s * _RPT, _RPT)])

        @pl.when(c == 0)
        def _():
            run(hL_hbm, sL_hbm, dL_hbm, outL_hbm)

        @pl.when(c == 1)
        def _():
            run(hG_hbm, sG_hbm, dG_hbm, outG_hbm)

    return scat_kernel(hL, hG, sLi, dLi, sGi, dGi)


_B = 1000


def _ln_relu(v, g, b):
    mu = jnp.mean(v, axis=-1, keepdims=True)
    d = v - mu
    var = jnp.mean(d * d, axis=-1, keepdims=True)
    return jnp.maximum(g * d * lax.rsqrt(var + 1e-5) + b, 0.0)


_row = pl.BlockSpec((_B, _H), lambda i: (i, 0))
_mat = pl.BlockSpec((_H, _H), lambda i: (0, 0))
_vec = pl.BlockSpec((1, _H), lambda i: (0, 0))


def _tc_pre(x, WL, WG, bL, bG, degLb, degGb):
    def body(x_ref, wl_ref, wg_ref, bl_ref, bg_ref, dl_ref, dg_ref,
             hLp_ref, hGp_ref, selfL_ref, selfG_ref):
        xb = x_ref[...]
        dl = lax.rsqrt(dl_ref[...] + 2.0)
        dg = lax.rsqrt(dg_ref[...] + 2.0)
        hl = jnp.dot(xb, wl_ref[...], preferred_element_type=jnp.float32)
        hg = jnp.dot(xb, wg_ref[...], preferred_element_type=jnp.float32)
        hLp_ref[...] = dl * hl
        hGp_ref[...] = dg * hg
        selfL_ref[...] = 2.0 * dl * dl * hl + bl_ref[...]
        selfG_ref[...] = 2.0 * dg * dg * hg + bg_ref[...]

    return pl.pallas_call(
        body,
        grid=(_N // _B,),
        in_specs=[_row, _mat, _mat, _vec, _vec, _row, _row],
        out_specs=[_row, _row, _row, _row],
        out_shape=[jax.ShapeDtypeStruct((_N, _H), jnp.float32)] * 4,
    )(x, WL, WG, bL, bG, degLb, degGb)


def _tc_mid(sL, sG, selfL, selfG, degLb, degGb, gm, bt,
            l0g, l0l, l0b, WL1, WG1, bL1, bG1):
    def body(sl_ref, sg_ref, fl_ref, fg_ref, dl_ref, dg_ref, gm_ref, bt_ref,
             l0g_ref, l0l_ref, l0b_ref, wl1_ref, wg1_ref, bl1_ref, bg1_ref,
             hL1p_ref, hG1p_ref, selfL1_ref, selfG1_ref, x1_ref):
        dl = lax.rsqrt(dl_ref[...] + 2.0)
        dg = lax.rsqrt(dg_ref[...] + 2.0)
        xL = _ln_relu(dl * sl_ref[...] + fl_ref[...], gm_ref[...], bt_ref[...])
        xG = _ln_relu(dg * sg_ref[...] + fg_ref[...], gm_ref[...], bt_ref[...])
        x1 = (jnp.dot(xG, l0g_ref[...], preferred_element_type=jnp.float32)
              + jnp.dot(xL, l0l_ref[...], preferred_element_type=jnp.float32)
              + l0b_ref[...])
        hl1 = jnp.dot(x1, wl1_ref[...], preferred_element_type=jnp.float32)
        hg1 = jnp.dot(x1, wg1_ref[...], preferred_element_type=jnp.float32)
        hL1p_ref[...] = dl * hl1
        hG1p_ref[...] = dg * hg1
        selfL1_ref[...] = 2.0 * dl * dl * hl1 + bl1_ref[...]
        selfG1_ref[...] = 2.0 * dg * dg * hg1 + bg1_ref[...]
        x1_ref[...] = x1

    return pl.pallas_call(
        body,
        grid=(_N // _B,),
        in_specs=[_row, _row, _row, _row, _row, _row, _vec, _vec,
                  _mat, _mat, _vec, _mat, _mat, _vec, _vec],
        out_specs=[_row, _row, _row, _row, _row],
        out_shape=[jax.ShapeDtypeStruct((_N, _H), jnp.float32)] * 5,
    )(sL, sG, selfL, selfG, degLb, degGb, gm, bt,
      l0g, l0l, l0b, WL1, WG1, bL1, bG1)


def _tc_post(sL, sG, selfL, selfG, degLb, degGb, x1, gm, bt,
             l1g, l1l, l1b, finW, finb):
    def body(sl_ref, sg_ref, fl_ref, fg_ref, dl_ref, dg_ref, x1_ref,
             gm_ref, bt_ref, l1g_ref, l1l_ref, l1b_ref, fw_ref, fb_ref,
             xf_ref, x2_ref):
        dl = lax.rsqrt(dl_ref[...] + 2.0)
        dg = lax.rsqrt(dg_ref[...] + 2.0)
        x1 = x1_ref[...]
        xL = _ln_relu(dl * sl_ref[...] + fl_ref[...],
                      gm_ref[...], bt_ref[...]) + x1
        xG = _ln_relu(dg * sg_ref[...] + fg_ref[...],
                      gm_ref[...], bt_ref[...]) + x1
        x2 = (jnp.dot(xG, l1g_ref[...], preferred_element_type=jnp.float32)
              + jnp.dot(xL, l1l_ref[...], preferred_element_type=jnp.float32)
              + l1b_ref[...])
        xf_ref[...] = (jnp.dot(x2, fw_ref[...],
                               preferred_element_type=jnp.float32)
                       + fb_ref[...])
        x2_ref[...] = x2

    return pl.pallas_call(
        body,
        grid=(_N // _B,),
        in_specs=[_row, _row, _row, _row, _row, _row, _row, _vec, _vec,
                  _mat, _mat, _vec, _mat, _vec],
        out_specs=[_row, _row],
        out_shape=[jax.ShapeDtypeStruct((_N, _H), jnp.float32)] * 2,
    )(sL, sG, selfL, selfG, degLb, degGb, x1, gm, bt,
      l1g, l1l, l1b, finW, finb)


def kernel(x, WL0, bL0, WG0, bG0, lin0W, lin0b, WL1, bL1, WG1, bG1,
           lin1W, lin1b, gamma, beta, finW, finb,
           G_edge_index, L_edge_index):
    degLr, degGr = _sc_degree(L_edge_index[1], G_edge_index[1])
    degLb = jnp.broadcast_to(degLr[:_N, None], (_N, _H))
    degGb = jnp.broadcast_to(degGr[:_N, None], (_N, _H))

    bL0r = bL0.reshape(1, _H)
    bG0r = bG0.reshape(1, _H)
    bL1r = bL1.reshape(1, _H)
    bG1r = bG1.reshape(1, _H)
    gm = gamma.reshape(1, _H)
    bt = beta.reshape(1, _H)
    l0b = lin0b.reshape(1, _H)
    l1b = lin1b.reshape(1, _H)
    fb = finb.reshape(1, _H)

    hLp, hGp, selfL, selfG = _tc_pre(x, WL0, WG0, bL0r, bG0r, degLb, degGb)
    srcL, dstL = L_edge_index[0], L_edge_index[1]
    srcG, dstG = G_edge_index[0], G_edge_index[1]
    sL, sG = _sc_gather_scatter(hLp, hGp, srcL, dstL, srcG, dstG)
    hL1p, hG1p, selfL1, selfG1, x1 = _tc_mid(
        sL[:_N], sG[:_N], selfL, selfG, degLb, degGb, gm, bt,
        lin0W[:_H], lin0W[_H:], l0b, WL1, WG1, bL1r, bG1r)
    sL1, sG1 = _sc_gather_scatter(hL1p, hG1p, srcL, dstL, srcG, dstG)
    xf, x2 = _tc_post(sL1[:_N], sG1[:_N], selfL1, selfG1, degLb, degGb, x1,
                      gm, bt, lin1W[:_H], lin1W[_H:], l1b, finW, fb)
    return (xf, x2)


# speedup vs baseline: 23.8632x; 1.6021x over previous
import functools

import jax
import jax.numpy as jnp
from jax import lax
from jax.experimental import pallas as pl
from jax.experimental.pallas import tpu as pltpu
from jax.experimental.pallas import tpu_sc as plsc

_N = 10000
_E = 320000
_H = 128
_NP = 10240
_NS = 16
_RPT = _NP // _NS
_EPT = _E // _NS
_CH = 125
_CHUNKS = _EPT // _CH
_BLK = 16
_NBLK = _CHUNKS // _BLK
_NSL = 4
_ZR = 80


def _sc_degree(dstL, dstG):
    mesh = plsc.VectorSubcoreMesh(core_axis_name="c", subcore_axis_name="s")

    @functools.partial(
        pl.kernel,
        out_type=(jax.ShapeDtypeStruct((_NP,), jnp.float32),
                  jax.ShapeDtypeStruct((_NP,), jnp.float32)),
        mesh=mesh,
        scratch_types=[
            pltpu.VMEM((_BLK, _CH), jnp.int32),
            pltpu.VMEM((_BLK, _CH), jnp.int32),
            pltpu.VMEM((128,), jnp.float32),
            pltpu.VMEM((_RPT,), jnp.float32),
            pltpu.VMEM_SHARED((_NP,), jnp.float32),
            pltpu.SemaphoreType.DMA,
            pltpu.SemaphoreType.DMA,
            pltpu.SemaphoreType.DMA,
            pltpu.SemaphoreType.DMA,
        ],
    )
    def deg_kernel(dstL_hbm, dstG_hbm, outL_hbm, outG_hbm,
                   blkA, blkB, ones_v, stage_v, deg_sh,
                   semLA, semLB, semWA, semWB):
        c = lax.axis_index("c")
        s = lax.axis_index("s")
        blk = (blkA, blkB)
        semL = (semLA, semLB)
        semW = (semWA, semWB)

        def run(dst_hbm, out_hbm):
            brow = s * (_EPT // _CH)

            def fill(j, carry):
                ones_v[pl.ds(j * 16, 16)] = jnp.ones((16,), jnp.float32)
                return carry

            lax.fori_loop(0, 8, fill, 0)

            def zfill(j, carry):
                stage_v[pl.ds(j * 16, 16)] = jnp.zeros((16,), jnp.float32)
                return carry

            lax.fori_loop(0, _RPT // 16, zfill, 0)
            pltpu.sync_copy(stage_v, deg_sh.at[pl.ds(s * _RPT, _RPT)])
            plsc.subcore_barrier()

            def load_block(i):
                buf = i % 2
                off = brow + i * _BLK
                pltpu.async_copy(
                    dst_hbm.at[pl.ds(off, _BLK)], blk[buf], semL[buf])

            def drain_scatters(buf):
                def dr(j, carry):
                    pltpu.make_async_copy(
                        ones_v.at[pl.ds(0, _CH)], deg_sh.at[blk[buf].at[0]],
                        semW[buf]).wait()
                    return carry
                lax.fori_loop(0, _BLK, dr, 0)

            load_block(0)
            for i in range(_NBLK):
                cur = i % 2
                oth = 1 - cur
                if i >= 1:
                    drain_scatters(oth)
                if i + 1 < _NBLK:
                    load_block(i + 1)
                pltpu.make_async_copy(
                    dst_hbm.at[pl.ds(brow, _BLK)], blk[cur], semL[cur]).wait()

                def body(jj, carry, cur=cur):
                    pltpu.async_copy(ones_v.at[pl.ds(0, _CH)],
                                     deg_sh.at[blk[cur].at[jj]],
                                     semW[cur], add=True)
                    return carry

                lax.fori_loop(0, _BLK, body, 0)
            drain_scatters((_NBLK - 1) % 2)
            plsc.subcore_barrier()
            pltpu.sync_copy(deg_sh.at[pl.ds(s * _RPT, _RPT)],
                            out_hbm.at[pl.ds(s * _RPT, _RPT)])

        @pl.when(c == 0)
        def _():
            run(dstL_hbm, outL_hbm)

        @pl.when(c == 1)
        def _():
            run(dstG_hbm, outG_hbm)

    return deg_kernel(dstL, dstG)


_CHS = 40
_NCH = _EPT // _CHS
_SL = 5
_NGRP = _NCH // _SL


def _sc_gather_scatter(hL, hG, sLi, dLi, sGi, dGi):
    mesh = plsc.VectorSubcoreMesh(core_axis_name="c", subcore_axis_name="s")

    @functools.partial(
        pl.kernel,
        out_type=(jax.ShapeDtypeStruct((_NP, _H), jnp.float32),
                  jax.ShapeDtypeStruct((_NP, _H), jnp.float32)),
        mesh=mesh,
        scratch_types=(
            [pltpu.VMEM((_CHS,), jnp.int32)] * _SL
            + [pltpu.VMEM((_CHS,), jnp.int32)] * (2 * _SL)
            + [pltpu.VMEM((_CHS, _H), jnp.float32)] * _SL
            + [pltpu.VMEM_SHARED((_NP, _H), jnp.float32)]
            + [pltpu.SemaphoreType.DMA] * (5 * _SL)
        ),
    )
    def scat_kernel(hL_hbm, hG_hbm, sL_hbm, dL_hbm, sG_hbm, dG_hbm,
                    outL_hbm, outG_hbm, *scr):
        srcb = scr[0:_SL]
        dstb = (scr[_SL:2 * _SL], scr[2 * _SL:3 * _SL])
        rows = scr[3 * _SL:4 * _SL]
        acc_sh = scr[4 * _SL]
        semS = scr[4 * _SL + 1:5 * _SL + 1]
        semD = (scr[5 * _SL + 1:6 * _SL + 1], scr[6 * _SL + 1:7 * _SL + 1])
        semG = scr[7 * _SL + 1:8 * _SL + 1]
        semW = scr[8 * _SL + 1:9 * _SL + 1]
        c = lax.axis_index("c")
        s = lax.axis_index("s")

        def run(h_hbm, se_hbm, de_hbm, out_hbm):
            base = s * _EPT

            def zrow(j, carry):
                rows[0][j // 8, pl.ds((j % 8) * 16, 16)] = (
                    jnp.zeros((16,), jnp.float32))
                return carry

            lax.fori_loop(0, _CHS * 8, zrow, 0)

            def zdma(j, carry):
                pltpu.sync_copy(rows[0],
                                acc_sh.at[pl.ds(s * _RPT + j * _CHS, _CHS)])
                return carry

            lax.fori_loop(0, _RPT // _CHS, zdma, 0)
            plsc.subcore_barrier()

            for b in range(_SL):
                off = base + b * _CHS
                pltpu.async_copy(se_hbm.at[pl.ds(off, _CHS)], srcb[b],
                                 semS[b])
                pltpu.async_copy(de_hbm.at[pl.ds(off, _CHS)], dstb[0][b],
                                 semD[0][b])

            def pair(i, carry):
                for p in range(2):
                    g = 2 * i + p
                    t0 = g * _SL * _CHS
                    for b in range(_SL):
                        @pl.when(g >= 1)
                        def _(b=b, p=p):
                            pltpu.make_async_copy(
                                rows[b], acc_sh.at[dstb[p][b]],
                                semW[b]).wait()
                        pltpu.make_async_copy(
                            se_hbm.at[pl.ds(base, _CHS)], srcb[b],
                            semS[b]).wait()
                        pltpu.async_copy(h_hbm.at[srcb[b]], rows[b], semG[b])
                    for b in range(_SL):
                        pltpu.make_async_copy(
                            h_hbm.at[srcb[b]], rows[b], semG[b]).wait()
                        pltpu.make_async_copy(
                            de_hbm.at[pl.ds(base, _CHS)], dstb[p][b],
                            semD[p][b]).wait()
                        pltpu.async_copy(rows[b], acc_sh.at[dstb[p][b]],
                                         semW[b], add=True)
                    for b in range(_SL):
                        @pl.when(g + 1 < _NGRP)
                        def _(b=b, p=p, t0=t0):
                            off = base + t0 + (_SL + b) * _CHS
                            pltpu.async_copy(se_hbm.at[pl.ds(off, _CHS)],
                                             srcb[b], semS[b])
                            pltpu.async_copy(de_hbm.at[pl.ds(off, _CHS)],
                                             dstb[1 - p][b], semD[1 - p][b])
                return carry

            lax.fori_loop(0, _NGRP // 2, pair, 0)
            for b in range(_SL):
                pltpu.make_async_copy(
                    rows[b], acc_sh.at[dstb[1][b]], semW[b]).wait()
            plsc.subcore_barrier()
            pltpu.sync_copy(acc_sh.at[pl.ds(s * _RPT, _RPT)],
                            out_hbm.at[pl.ds(s * _RPT, _RPT)])

        @pl.when(c == 0)
        def _():
            run(hL_hbm, sL_hbm, dL_hbm, outL_hbm)

        @pl.when(c == 1)
        def _():
            run(hG_hbm, sG_hbm, dG_hbm, outG_hbm)

    return scat_kernel(hL, hG, sLi, dLi, sGi, dGi)


_B = 1000


def _ln_relu(v, g, b):
    mu = jnp.mean(v, axis=-1, keepdims=True)
    d = v - mu
    var = jnp.mean(d * d, axis=-1, keepdims=True)
    return jnp.maximum(g * d * lax.rsqrt(var + 1e-5) + b, 0.0)


_row = pl.BlockSpec((_B, _H), lambda i: (i, 0))
_mat = pl.BlockSpec((_H, _H), lambda i: (0, 0))
_vec = pl.BlockSpec((1, _H), lambda i: (0, 0))


def _tc_pre(x, WL, WG, bL, bG, degLb, degGb):
    def body(x_ref, wl_ref, wg_ref, bl_ref, bg_ref, dl_ref, dg_ref,
             hLp_ref, hGp_ref, selfL_ref, selfG_ref):
        xb = x_ref[...]
        dl = lax.rsqrt(dl_ref[...] + 2.0)
        dg = lax.rsqrt(dg_ref[...] + 2.0)
        hl = jnp.dot(xb, wl_ref[...], preferred_element_type=jnp.float32)
        hg = jnp.dot(xb, wg_ref[...], preferred_element_type=jnp.float32)
        hLp_ref[...] = dl * hl
        hGp_ref[...] = dg * hg
        selfL_ref[...] = 2.0 * dl * dl * hl + bl_ref[...]
        selfG_ref[...] = 2.0 * dg * dg * hg + bg_ref[...]

    return pl.pallas_call(
        body,
        grid=(_N // _B,),
        in_specs=[_row, _mat, _mat, _vec, _vec, _row, _row],
        out_specs=[_row, _row, _row, _row],
        out_shape=[jax.ShapeDtypeStruct((_N, _H), jnp.float32)] * 4,
    )(x, WL, WG, bL, bG, degLb, degGb)


def _tc_mid(sL, sG, selfL, selfG, degLb, degGb, gm, bt,
            l0g, l0l, l0b, WL1, WG1, bL1, bG1):
    def body(sl_ref, sg_ref, fl_ref, fg_ref, dl_ref, dg_ref, gm_ref, bt_ref,
             l0g_ref, l0l_ref, l0b_ref, wl1_ref, wg1_ref, bl1_ref, bg1_ref,
             hL1p_ref, hG1p_ref, selfL1_ref, selfG1_ref, x1_ref):
        dl = lax.rsqrt(dl_ref[...] + 2.0)
        dg = lax.rsqrt(dg_ref[...] + 2.0)
        xL = _ln_relu(dl * sl_ref[...] + fl_ref[...], gm_ref[...], bt_ref[...])
        xG = _ln_relu(dg * sg_ref[...] + fg_ref[...], gm_ref[...], bt_ref[...])
        x1 = (jnp.dot(xG, l0g_ref[...], preferred_element_type=jnp.float32)
              + jnp.dot(xL, l0l_ref[...], preferred_element_type=jnp.float32)
              + l0b_ref[...])
        hl1 = jnp.dot(x1, wl1_ref[...], preferred_element_type=jnp.float32)
        hg1 = jnp.dot(x1, wg1_ref[...], preferred_element_type=jnp.float32)
        hL1p_ref[...] = dl * hl1
        hG1p_ref[...] = dg * hg1
        selfL1_ref[...] = 2.0 * dl * dl * hl1 + bl1_ref[...]
        selfG1_ref[...] = 2.0 * dg * dg * hg1 + bg1_ref[...]
        x1_ref[...] = x1

    return pl.pallas_call(
        body,
        grid=(_N // _B,),
        in_specs=[_row, _row, _row, _row, _row, _row, _vec, _vec,
                  _mat, _mat, _vec, _mat, _mat, _vec, _vec],
        out_specs=[_row, _row, _row, _row, _row],
        out_shape=[jax.ShapeDtypeStruct((_N, _H), jnp.float32)] * 5,
    )(sL, sG, selfL, selfG, degLb, degGb, gm, bt,
      l0g, l0l, l0b, WL1, WG1, bL1, bG1)


def _tc_post(sL, sG, selfL, selfG, degLb, degGb, x1, gm, bt,
             l1g, l1l, l1b, finW, finb):
    def body(sl_ref, sg_ref, fl_ref, fg_ref, dl_ref, dg_ref, x1_ref,
             gm_ref, bt_ref, l1g_ref, l1l_ref, l1b_ref, fw_ref, fb_ref,
             xf_ref, x2_ref):
        dl = lax.rsqrt(dl_ref[...] + 2.0)
        dg = lax.rsqrt(dg_ref[...] + 2.0)
        x1 = x1_ref[...]
        xL = _ln_relu(dl * sl_ref[...] + fl_ref[...],
                      gm_ref[...], bt_ref[...]) + x1
        xG = _ln_relu(dg * sg_ref[...] + fg_ref[...],
                      gm_ref[...], bt_ref[...]) + x1
        x2 = (jnp.dot(xG, l1g_ref[...], preferred_element_type=jnp.float32)
              + jnp.dot(xL, l1l_ref[...], preferred_element_type=jnp.float32)
              + l1b_ref[...])
        xf_ref[...] = (jnp.dot(x2, fw_ref[...],
                               preferred_element_type=jnp.float32)
                       + fb_ref[...])
        x2_ref[...] = x2

    return pl.pallas_call(
        body,
        grid=(_N // _B,),
        in_specs=[_row, _row, _row, _row, _row, _row, _row, _vec, _vec,
                  _mat, _mat, _vec, _mat, _vec],
        out_specs=[_row, _row],
        out_shape=[jax.ShapeDtypeStruct((_N, _H), jnp.float32)] * 2,
    )(sL, sG, selfL, selfG, degLb, degGb, x1, gm, bt,
      l1g, l1l, l1b, finW, finb)


def kernel(x, WL0, bL0, WG0, bG0, lin0W, lin0b, WL1, bL1, WG1, bG1,
           lin1W, lin1b, gamma, beta, finW, finb,
           G_edge_index, L_edge_index):
    degLr, degGr = _sc_degree(L_edge_index[1].reshape(_E // _CH, _CH),
                              G_edge_index[1].reshape(_E // _CH, _CH))
    degLb = jnp.broadcast_to(degLr[:_N, None], (_N, _H))
    degGb = jnp.broadcast_to(degGr[:_N, None], (_N, _H))

    bL0r = bL0.reshape(1, _H)
    bG0r = bG0.reshape(1, _H)
    bL1r = bL1.reshape(1, _H)
    bG1r = bG1.reshape(1, _H)
    gm = gamma.reshape(1, _H)
    bt = beta.reshape(1, _H)
    l0b = lin0b.reshape(1, _H)
    l1b = lin1b.reshape(1, _H)
    fb = finb.reshape(1, _H)

    hLp, hGp, selfL, selfG = _tc_pre(x, WL0, WG0, bL0r, bG0r, degLb, degGb)
    srcL, dstL = L_edge_index[0], L_edge_index[1]
    srcG, dstG = G_edge_index[0], G_edge_index[1]
    sL, sG = _sc_gather_scatter(hLp, hGp, srcL, dstL, srcG, dstG)
    hL1p, hG1p, selfL1, selfG1, x1 = _tc_mid(
        sL[:_N], sG[:_N], selfL, selfG, degLb, degGb, gm, bt,
        lin0W[:_H], lin0W[_H:], l0b, WL1, WG1, bL1r, bG1r)
    sL1, sG1 = _sc_gather_scatter(hL1p, hG1p, srcL, dstL, srcG, dstG)
    xf, x2 = _tc_post(sL1[:_N], sG1[:_N], selfL1, selfG1, degLb, degGb, x1,
                      gm, bt, lin1W[:_H], lin1W[_H:], l1b, finW, fb)
    return (xf, x2)


# speedup vs baseline: 25.0717x; 1.0506x over previous
import functools

import jax
import jax.numpy as jnp
from jax import lax
from jax.experimental import pallas as pl
from jax.experimental.pallas import tpu as pltpu
from jax.experimental.pallas import tpu_sc as plsc

_N = 10000
_E = 320000
_H = 128
_NP = 10240
_NS = 16
_RPT = _NP // _NS
_EPT = _E // _NS
_CH = 125
_CHUNKS = _EPT // _CH
_BLK = 16
_NBLK = _CHUNKS // _BLK
_NSL = 4
_ZR = 80


def _sc_degree(dstL, dstG):
    mesh = plsc.VectorSubcoreMesh(core_axis_name="c", subcore_axis_name="s")

    @functools.partial(
        pl.kernel,
        out_type=(jax.ShapeDtypeStruct((_NP,), jnp.float32),
                  jax.ShapeDtypeStruct((_NP,), jnp.float32)),
        mesh=mesh,
        scratch_types=[
            pltpu.VMEM((_BLK, _CH), jnp.int32),
            pltpu.VMEM((_BLK, _CH), jnp.int32),
            pltpu.VMEM((128,), jnp.float32),
            pltpu.VMEM((_RPT,), jnp.float32),
            pltpu.VMEM_SHARED((_NP,), jnp.float32),
            pltpu.SemaphoreType.DMA,
            pltpu.SemaphoreType.DMA,
            pltpu.SemaphoreType.DMA,
            pltpu.SemaphoreType.DMA,
        ],
    )
    def deg_kernel(dstL_hbm, dstG_hbm, outL_hbm, outG_hbm,
                   blkA, blkB, ones_v, stage_v, deg_sh,
                   semLA, semLB, semWA, semWB):
        c = lax.axis_index("c")
        s = lax.axis_index("s")
        blk = (blkA, blkB)
        semL = (semLA, semLB)
        semW = (semWA, semWB)

        def run(dst_hbm, out_hbm):
            brow = s * (_EPT // _CH)

            def fill(j, carry):
                ones_v[pl.ds(j * 16, 16)] = jnp.ones((16,), jnp.float32)
                return carry

            lax.fori_loop(0, 8, fill, 0)

            def zfill(j, carry):
                stage_v[pl.ds(j * 16, 16)] = jnp.zeros((16,), jnp.float32)
                return carry

            lax.fori_loop(0, _RPT // 16, zfill, 0)
            pltpu.sync_copy(stage_v, deg_sh.at[pl.ds(s * _RPT, _RPT)])
            plsc.subcore_barrier()

            def load_block(i):
                buf = i % 2
                off = brow + i * _BLK
                pltpu.async_copy(
                    dst_hbm.at[pl.ds(off, _BLK)], blk[buf], semL[buf])

            def drain_scatters(buf):
                def dr(j, carry):
                    pltpu.make_async_copy(
                        ones_v.at[pl.ds(0, _CH)], deg_sh.at[blk[buf].at[0]],
                        semW[buf]).wait()
                    return carry
                lax.fori_loop(0, _BLK, dr, 0)

            load_block(0)
            for i in range(_NBLK):
                cur = i % 2
                oth = 1 - cur
                if i >= 1:
                    drain_scatters(oth)
                if i + 1 < _NBLK:
                    load_block(i + 1)
                pltpu.make_async_copy(
                    dst_hbm.at[pl.ds(brow, _BLK)], blk[cur], semL[cur]).wait()

                def body(jj, carry, cur=cur):
                    pltpu.async_copy(ones_v.at[pl.ds(0, _CH)],
                                     deg_sh.at[blk[cur].at[jj]],
                                     semW[cur], add=True)
                    return carry

                lax.fori_loop(0, _BLK, body, 0)
            drain_scatters((_NBLK - 1) % 2)
            plsc.subcore_barrier()
            pltpu.sync_copy(deg_sh.at[pl.ds(s * _RPT, _RPT)],
                            out_hbm.at[pl.ds(s * _RPT, _RPT)])

        @pl.when(c == 0)
        def _():
            run(dstL_hbm, outL_hbm)

        @pl.when(c == 1)
        def _():
            run(dstG_hbm, outG_hbm)

    return deg_kernel(dstL, dstG)


_CHS = 64
_SL = 4
_TCH = 312
_TBASE = _TCH * _CHS


def _sc_gather_scatter(hL, hG, sLi, dLi, sGi, dGi):
    mesh = plsc.VectorSubcoreMesh(core_axis_name="c", subcore_axis_name="s")

    @functools.partial(
        pl.kernel,
        out_type=(jax.ShapeDtypeStruct((_NP, _H), jnp.float32),
                  jax.ShapeDtypeStruct((_NP, _H), jnp.float32)),
        mesh=mesh,
        scratch_types=(
            [pltpu.VMEM((_CHS,), jnp.int32)] * _SL
            + [pltpu.VMEM((_CHS,), jnp.int32)] * (2 * _SL)
            + [pltpu.VMEM((_CHS, _H), jnp.float32)] * _SL
            + [pltpu.VMEM_SHARED((_NP, _H), jnp.float32)]
            + [pltpu.SemaphoreType.DMA] * (5 * _SL)
        ),
    )
    def scat_kernel(hL_hbm, hG_hbm, sL_hbm, dL_hbm, sG_hbm, dG_hbm,
                    outL_hbm, outG_hbm, *scr):
        srcb = scr[0:_SL]
        dstb = (scr[_SL:2 * _SL], scr[2 * _SL:3 * _SL])
        rows = scr[3 * _SL:4 * _SL]
        acc_sh = scr[4 * _SL]
        semS = scr[4 * _SL + 1:5 * _SL + 1]
        semD = (scr[5 * _SL + 1:6 * _SL + 1], scr[6 * _SL + 1:7 * _SL + 1])
        semG = scr[7 * _SL + 1:8 * _SL + 1]
        semW = scr[8 * _SL + 1:9 * _SL + 1]
        c = lax.axis_index("c")
        s = lax.axis_index("s")

        def run(h_hbm, se_hbm, de_hbm, out_hbm):
            base = s * _TBASE
            ngrp = jnp.where(s == _NS - 1, 80, 78)

            def zrow(j, carry):
                rows[0][j // 8, pl.ds((j % 8) * 16, 16)] = (
                    jnp.zeros((16,), jnp.float32))
                return carry

            lax.fori_loop(0, _CHS * 8, zrow, 0)

            def zdma(j, carry):
                pltpu.sync_copy(rows[0],
                                acc_sh.at[pl.ds(s * _RPT + j * _CHS, _CHS)])
                return carry

            lax.fori_loop(0, _RPT // _CHS, zdma, 0)
            assert _RPT % _CHS == 0
            plsc.subcore_barrier()

            for b in range(_SL):
                off = base + b * _CHS
                pltpu.async_copy(se_hbm.at[pl.ds(off, _CHS)], srcb[b],
                                 semS[b])
                pltpu.async_copy(de_hbm.at[pl.ds(off, _CHS)], dstb[0][b],
                                 semD[0][b])

            def pair(i, carry):
                for p in range(2):
                    g = 2 * i + p
                    t0 = g * _SL * _CHS
                    for b in range(_SL):
                        @pl.when(g >= 1)
                        def _(b=b, p=p):
                            pltpu.make_async_copy(
                                rows[b], acc_sh.at[dstb[p][b]],
                                semW[b]).wait()
                        pltpu.make_async_copy(
                            se_hbm.at[pl.ds(base, _CHS)], srcb[b],
                            semS[b]).wait()
                        pltpu.async_copy(h_hbm.at[srcb[b]], rows[b], semG[b])
                    for b in range(_SL):
                        pltpu.make_async_copy(
                            h_hbm.at[srcb[b]], rows[b], semG[b]).wait()
                        pltpu.make_async_copy(
                            de_hbm.at[pl.ds(base, _CHS)], dstb[p][b],
                            semD[p][b]).wait()
                        pltpu.async_copy(rows[b], acc_sh.at[dstb[p][b]],
                                         semW[b], add=True)
                    for b in range(_SL):
                        @pl.when(g + 1 < ngrp)
                        def _(b=b, p=p, t0=t0):
                            off = base + t0 + (_SL + b) * _CHS
                            pltpu.async_copy(se_hbm.at[pl.ds(off, _CHS)],
                                             srcb[b], semS[b])
                            pltpu.async_copy(de_hbm.at[pl.ds(off, _CHS)],
                                             dstb[1 - p][b], semD[1 - p][b])
                return carry

            lax.fori_loop(0, ngrp // 2, pair, 0)
            for b in range(_SL):
                pltpu.make_async_copy(
                    rows[b], acc_sh.at[dstb[1][b]], semW[b]).wait()
            plsc.subcore_barrier()
            pltpu.sync_copy(acc_sh.at[pl.ds(s * _RPT, _RPT)],
                            out_hbm.at[pl.ds(s * _RPT, _RPT)])

        @pl.when(c == 0)
        def _():
            run(hL_hbm, sL_hbm, dL_hbm, outL_hbm)

        @pl.when(c == 1)
        def _():
            run(hG_hbm, sG_hbm, dG_hbm, outG_hbm)

    return scat_kernel(hL, hG, sLi, dLi, sGi, dGi)


_B = 1000


def _ln_relu(v, g, b):
    mu = jnp.mean(v, axis=-1, keepdims=True)
    d = v - mu
    var = jnp.mean(d * d, axis=-1, keepdims=True)
    return jnp.maximum(g * d * lax.rsqrt(var + 1e-5) + b, 0.0)


_row = pl.BlockSpec((_B, _H), lambda i: (i, 0))
_mat = pl.BlockSpec((_H, _H), lambda i: (0, 0))
_vec = pl.BlockSpec((1, _H), lambda i: (0, 0))


def _tc_pre(x, WL, WG, bL, bG, degLb, degGb):
    def body(x_ref, wl_ref, wg_ref, bl_ref, bg_ref, dl_ref, dg_ref,
             hLp_ref, hGp_ref, selfL_ref, selfG_ref):
        xb = x_ref[...]
        dl = lax.rsqrt(dl_ref[...] + 2.0)
        dg = lax.rsqrt(dg_ref[...] + 2.0)
        hl = jnp.dot(xb, wl_ref[...], preferred_element_type=jnp.float32)
        hg = jnp.dot(xb, wg_ref[...], preferred_element_type=jnp.float32)
        hLp_ref[...] = dl * hl
        hGp_ref[...] = dg * hg
        selfL_ref[...] = 2.0 * dl * dl * hl + bl_ref[...]
        selfG_ref[...] = 2.0 * dg * dg * hg + bg_ref[...]

    return pl.pallas_call(
        body,
        grid=(_N // _B,),
        in_specs=[_row, _mat, _mat, _vec, _vec, _row, _row],
        out_specs=[_row, _row, _row, _row],
        out_shape=[jax.ShapeDtypeStruct((_N, _H), jnp.float32)] * 4,
    )(x, WL, WG, bL, bG, degLb, degGb)


def _tc_mid(sL, sG, selfL, selfG, degLb, degGb, gm, bt,
            l0g, l0l, l0b, WL1, WG1, bL1, bG1):
    def body(sl_ref, sg_ref, fl_ref, fg_ref, dl_ref, dg_ref, gm_ref, bt_ref,
             l0g_ref, l0l_ref, l0b_ref, wl1_ref, wg1_ref, bl1_ref, bg1_ref,
             hL1p_ref, hG1p_ref, selfL1_ref, selfG1_ref, x1_ref):
        dl = lax.rsqrt(dl_ref[...] + 2.0)
        dg = lax.rsqrt(dg_ref[...] + 2.0)
        xL = _ln_relu(dl * sl_ref[...] + fl_ref[...], gm_ref[...], bt_ref[...])
        xG = _ln_relu(dg * sg_ref[...] + fg_ref[...], gm_ref[...], bt_ref[...])
        x1 = (jnp.dot(xG, l0g_ref[...], preferred_element_type=jnp.float32)
              + jnp.dot(xL, l0l_ref[...], preferred_element_type=jnp.float32)
              + l0b_ref[...])
        hl1 = jnp.dot(x1, wl1_ref[...], preferred_element_type=jnp.float32)
        hg1 = jnp.dot(x1, wg1_ref[...], preferred_element_type=jnp.float32)
        hL1p_ref[...] = dl * hl1
        hG1p_ref[...] = dg * hg1
        selfL1_ref[...] = 2.0 * dl * dl * hl1 + bl1_ref[...]
        selfG1_ref[...] = 2.0 * dg * dg * hg1 + bg1_ref[...]
        x1_ref[...] = x1

    return pl.pallas_call(
        body,
        grid=(_N // _B,),
        in_specs=[_row, _row, _row, _row, _row, _row, _vec, _vec,
                  _mat, _mat, _vec, _mat, _mat, _vec, _vec],
        out_specs=[_row, _row, _row, _row, _row],
        out_shape=[jax.ShapeDtypeStruct((_N, _H), jnp.float32)] * 5,
    )(sL, sG, selfL, selfG, degLb, degGb, gm, bt,
      l0g, l0l, l0b, WL1, WG1, bL1, bG1)


def _tc_post(sL, sG, selfL, selfG, degLb, degGb, x1, gm, bt,
             l1g, l1l, l1b, finW, finb):
    def body(sl_ref, sg_ref, fl_ref, fg_ref, dl_ref, dg_ref, x1_ref,
             gm_ref, bt_ref, l1g_ref, l1l_ref, l1b_ref, fw_ref, fb_ref,
             xf_ref, x2_ref):
        dl = lax.rsqrt(dl_ref[...] + 2.0)
        dg = lax.rsqrt(dg_ref[...] + 2.0)
        x1 = x1_ref[...]
        xL = _ln_relu(dl * sl_ref[...] + fl_ref[...],
                      gm_ref[...], bt_ref[...]) + x1
        xG = _ln_relu(dg * sg_ref[...] + fg_ref[...],
                      gm_ref[...], bt_ref[...]) + x1
        x2 = (jnp.dot(xG, l1g_ref[...], preferred_element_type=jnp.float32)
              + jnp.dot(xL, l1l_ref[...], preferred_element_type=jnp.float32)
              + l1b_ref[...])
        xf_ref[...] = (jnp.dot(x2, fw_ref[...],
                               preferred_element_type=jnp.float32)
                       + fb_ref[...])
        x2_ref[...] = x2

    return pl.pallas_call(
        body,
        grid=(_N // _B,),
        in_specs=[_row, _row, _row, _row, _row, _row, _row, _vec, _vec,
                  _mat, _mat, _vec, _mat, _vec],
        out_specs=[_row, _row],
        out_shape=[jax.ShapeDtypeStruct((_N, _H), jnp.float32)] * 2,
    )(sL, sG, selfL, selfG, degLb, degGb, x1, gm, bt,
      l1g, l1l, l1b, finW, finb)


def kernel(x, WL0, bL0, WG0, bG0, lin0W, lin0b, WL1, bL1, WG1, bG1,
           lin1W, lin1b, gamma, beta, finW, finb,
           G_edge_index, L_edge_index):
    degLr, degGr = _sc_degree(L_edge_index[1].reshape(_E // _CH, _CH),
                              G_edge_index[1].reshape(_E // _CH, _CH))
    degLb = jnp.broadcast_to(degLr[:_N, None], (_N, _H))
    degGb = jnp.broadcast_to(degGr[:_N, None], (_N, _H))

    bL0r = bL0.reshape(1, _H)
    bG0r = bG0.reshape(1, _H)
    bL1r = bL1.reshape(1, _H)
    bG1r = bG1.reshape(1, _H)
    gm = gamma.reshape(1, _H)
    bt = beta.reshape(1, _H)
    l0b = lin0b.reshape(1, _H)
    l1b = lin1b.reshape(1, _H)
    fb = finb.reshape(1, _H)

    hLp, hGp, selfL, selfG = _tc_pre(x, WL0, WG0, bL0r, bG0r, degLb, degGb)
    srcL, dstL = L_edge_index[0], L_edge_index[1]
    srcG, dstG = G_edge_index[0], G_edge_index[1]
    sL, sG = _sc_gather_scatter(hLp, hGp, srcL, dstL, srcG, dstG)
    hL1p, hG1p, selfL1, selfG1, x1 = _tc_mid(
        sL[:_N], sG[:_N], selfL, selfG, degLb, degGb, gm, bt,
        lin0W[:_H], lin0W[_H:], l0b, WL1, WG1, bL1r, bG1r)
    sL1, sG1 = _sc_gather_scatter(hL1p, hG1p, srcL, dstL, srcG, dstG)
    xf, x2 = _tc_post(sL1[:_N], sG1[:_N], selfL1, selfG1, degLb, degGb, x1,
                      gm, bt, lin1W[:_H], lin1W[_H:], l1b, finW, fb)
    return (xf, x2)


# speedup vs baseline: 25.3211x; 1.0099x over previous
import functools

import jax
import jax.numpy as jnp
from jax import lax
from jax.experimental import pallas as pl
from jax.experimental.pallas import tpu as pltpu
from jax.experimental.pallas import tpu_sc as plsc

_N = 10000
_E = 320000
_H = 128
_NP = 10240
_NS = 16
_RPT = _NP // _NS
_EPT = _E // _NS
_CH = 125
_CHUNKS = _EPT // _CH
_BLK = 16
_NBLK = _CHUNKS // _BLK
_NSL = 4
_ZR = 80


def _sc_degree(dstL, dstG):
    mesh = plsc.VectorSubcoreMesh(core_axis_name="c", subcore_axis_name="s")

    @functools.partial(
        pl.kernel,
        out_type=(jax.ShapeDtypeStruct((_NP,), jnp.float32),
                  jax.ShapeDtypeStruct((_NP,), jnp.float32)),
        mesh=mesh,
        scratch_types=[
            pltpu.VMEM((_BLK, _CH), jnp.int32),
            pltpu.VMEM((_BLK, _CH), jnp.int32),
            pltpu.VMEM((128,), jnp.float32),
            pltpu.VMEM((_RPT,), jnp.float32),
            pltpu.VMEM_SHARED((_NP,), jnp.float32),
            pltpu.SemaphoreType.DMA,
            pltpu.SemaphoreType.DMA,
            pltpu.SemaphoreType.DMA,
            pltpu.SemaphoreType.DMA,
        ],
    )
    def deg_kernel(dstL_hbm, dstG_hbm, outL_hbm, outG_hbm,
                   blkA, blkB, ones_v, stage_v, deg_sh,
                   semLA, semLB, semWA, semWB):
        c = lax.axis_index("c")
        s = lax.axis_index("s")
        blk = (blkA, blkB)
        semL = (semLA, semLB)
        semW = (semWA, semWB)

        def run(dst_hbm, out_hbm):
            brow = s * (_EPT // _CH)

            def fill(j, carry):
                ones_v[pl.ds(j * 16, 16)] = jnp.ones((16,), jnp.float32)
                return carry

            lax.fori_loop(0, 8, fill, 0)

            def zfill(j, carry):
                stage_v[pl.ds(j * 16, 16)] = jnp.zeros((16,), jnp.float32)
                return carry

            lax.fori_loop(0, _RPT // 16, zfill, 0)
            pltpu.sync_copy(stage_v, deg_sh.at[pl.ds(s * _RPT, _RPT)])
            plsc.subcore_barrier()

            def load_block(i):
                buf = i % 2
                off = brow + i * _BLK
                pltpu.async_copy(
                    dst_hbm.at[pl.ds(off, _BLK)], blk[buf], semL[buf])

            def drain_scatters(buf):
                def dr(j, carry):
                    pltpu.make_async_copy(
                        ones_v.at[pl.ds(0, _CH)], deg_sh.at[blk[buf].at[0]],
                        semW[buf]).wait()
                    return carry
                lax.fori_loop(0, _BLK, dr, 0)

            load_block(0)
            for i in range(_NBLK):
                cur = i % 2
                oth = 1 - cur
                if i >= 1:
                    drain_scatters(oth)
                if i + 1 < _NBLK:
                    load_block(i + 1)
                pltpu.make_async_copy(
                    dst_hbm.at[pl.ds(brow, _BLK)], blk[cur], semL[cur]).wait()

                def body(jj, carry, cur=cur):
                    pltpu.async_copy(ones_v.at[pl.ds(0, _CH)],
                                     deg_sh.at[blk[cur].at[jj]],
                                     semW[cur], add=True)
                    return carry

                lax.fori_loop(0, _BLK, body, 0)
            drain_scatters((_NBLK - 1) % 2)
            plsc.subcore_barrier()
            pltpu.sync_copy(deg_sh.at[pl.ds(s * _RPT, _RPT)],
                            out_hbm.at[pl.ds(s * _RPT, _RPT)])

        @pl.when(c == 0)
        def _():
            run(dstL_hbm, outL_hbm)

        @pl.when(c == 1)
        def _():
            run(dstG_hbm, outG_hbm)

    return deg_kernel(dstL, dstG)


_CHS = 64
_SL = 4
_TCH = 312
_TBASE = _TCH * _CHS


def _sc_gather_scatter(hL, hG, sLi, dLi, sGi, dGi):
    mesh = plsc.VectorSubcoreMesh(core_axis_name="c", subcore_axis_name="s")

    @functools.partial(
        pl.kernel,
        out_type=(jax.ShapeDtypeStruct((_NP, _H), jnp.float32),
                  jax.ShapeDtypeStruct((_NP, _H), jnp.float32)),
        mesh=mesh,
        scratch_types=(
            [pltpu.VMEM((_CHS,), jnp.int32)] * _SL
            + [pltpu.VMEM((_CHS,), jnp.int32)] * (2 * _SL)
            + [pltpu.VMEM((_CHS, _H), jnp.float32)] * _SL
            + [pltpu.VMEM_SHARED((_NP, _H), jnp.float32)]
            + [pltpu.SemaphoreType.DMA] * (5 * _SL)
        ),
    )
    def scat_kernel(hL_hbm, hG_hbm, sL_hbm, dL_hbm, sG_hbm, dG_hbm,
                    outL_hbm, outG_hbm, *scr):
        srcb = scr[0:_SL]
        dstb = (scr[_SL:2 * _SL], scr[2 * _SL:3 * _SL])
        rows = scr[3 * _SL:4 * _SL]
        acc_sh = scr[4 * _SL]
        semS = scr[4 * _SL + 1:5 * _SL + 1]
        semD = (scr[5 * _SL + 1:6 * _SL + 1], scr[6 * _SL + 1:7 * _SL + 1])
        semG = scr[7 * _SL + 1:8 * _SL + 1]
        semW = scr[8 * _SL + 1:9 * _SL + 1]
        c = lax.axis_index("c")
        s = lax.axis_index("s")

        def run(h_hbm, se_hbm, de_hbm, out_hbm):
            base = s * _TBASE
            ngrp = jnp.where(s == _NS - 1, 80, 78)

            def zrow(j, carry):
                rows[0][j // 8, pl.ds((j % 8) * 16, 16)] = (
                    jnp.zeros((16,), jnp.float32))
                return carry

            lax.fori_loop(0, _CHS * 8, zrow, 0)

            def zdma(j, carry):
                pltpu.sync_copy(rows[0],
                                acc_sh.at[pl.ds(s * _RPT + j * _CHS, _CHS)])
                return carry

            lax.fori_loop(0, _RPT // _CHS, zdma, 0)
            assert _RPT % _CHS == 0
            plsc.subcore_barrier()

            for b in range(_SL):
                off = base + b * _CHS
                pltpu.async_copy(se_hbm.at[pl.ds(off, _CHS)], srcb[b],
                                 semS[b])
                pltpu.async_copy(de_hbm.at[pl.ds(off, _CHS)], dstb[0][b],
                                 semD[0][b])

            def pair(i, carry):
                for p in range(2):
                    g = 2 * i + p
                    t0 = g * _SL * _CHS
                    for b in range(_SL):
                        @pl.when(g >= 1)
                        def _(b=b, p=p):
                            pltpu.make_async_copy(
                                rows[b], acc_sh.at[dstb[p][b]],
                                semW[b]).wait()
                        pltpu.make_async_copy(
                            se_hbm.at[pl.ds(base, _CHS)], srcb[b],
                            semS[b]).wait()
                        pltpu.async_copy(h_hbm.at[srcb[b]], rows[b], semG[b])
                    for b in range(_SL):
                        pltpu.make_async_copy(
                            h_hbm.at[srcb[b]], rows[b], semG[b]).wait()
                        pltpu.make_async_copy(
                            de_hbm.at[pl.ds(base, _CHS)], dstb[p][b],
                            semD[p][b]).wait()
                        pltpu.async_copy(rows[b], acc_sh.at[dstb[p][b]],
                                         semW[b], add=True)
                    for b in range(_SL):
                        @pl.when(g + 1 < ngrp)
                        def _(b=b, p=p, t0=t0):
                            off = base + t0 + (_SL + b) * _CHS
                            pltpu.async_copy(se_hbm.at[pl.ds(off, _CHS)],
                                             srcb[b], semS[b])
                            pltpu.async_copy(de_hbm.at[pl.ds(off, _CHS)],
                                             dstb[1 - p][b], semD[1 - p][b])
                return carry

            lax.fori_loop(0, ngrp // 2, pair, 0)
            for b in range(_SL):
                pltpu.make_async_copy(
                    rows[b], acc_sh.at[dstb[1][b]], semW[b]).wait()
            plsc.subcore_barrier()
            pltpu.sync_copy(acc_sh.at[pl.ds(s * _RPT, _RPT)],
                            out_hbm.at[pl.ds(s * _RPT, _RPT)])

        @pl.when(c == 0)
        def _():
            run(hL_hbm, sL_hbm, dL_hbm, outL_hbm)

        @pl.when(c == 1)
        def _():
            run(hG_hbm, sG_hbm, dG_hbm, outG_hbm)

    return scat_kernel(hL, hG, sLi, dLi, sGi, dGi)


_B = 1024


def _ln_relu(v, g, b):
    mu = jnp.mean(v, axis=-1, keepdims=True)
    d = v - mu
    var = jnp.mean(d * d, axis=-1, keepdims=True)
    return jnp.maximum(g * d * lax.rsqrt(var + 1e-5) + b, 0.0)


_row = pl.BlockSpec((_B, _H), lambda i: (i, 0))
_mat = pl.BlockSpec((_H, _H), lambda i: (0, 0))
_vec = pl.BlockSpec((1, _H), lambda i: (0, 0))
_deg = pl.BlockSpec((_B, 1), lambda i: (i, 0))


def _tc_pre(x, WL, WG, bL, bG, degLb, degGb):
    def body(x_ref, wl_ref, wg_ref, bl_ref, bg_ref, dl_ref, dg_ref,
             hLp_ref, hGp_ref, selfL_ref, selfG_ref):
        xb = x_ref[...]
        dl = lax.rsqrt(dl_ref[...] + 2.0)
        dg = lax.rsqrt(dg_ref[...] + 2.0)
        hl = jnp.dot(xb, wl_ref[...], preferred_element_type=jnp.float32)
        hg = jnp.dot(xb, wg_ref[...], preferred_element_type=jnp.float32)
        hLp_ref[...] = dl * hl
        hGp_ref[...] = dg * hg
        selfL_ref[...] = 2.0 * dl * dl * hl + bl_ref[...]
        selfG_ref[...] = 2.0 * dg * dg * hg + bg_ref[...]

    return pl.pallas_call(
        body,
        grid=(_NP // _B,),
        in_specs=[_row, _mat, _mat, _vec, _vec, _deg, _deg],
        out_specs=[_row, _row, _row, _row],
        out_shape=[jax.ShapeDtypeStruct((_NP, _H), jnp.float32)] * 4,
    )(x, WL, WG, bL, bG, degLb, degGb)


def _tc_mid(sL, sG, selfL, selfG, degLb, degGb, gm, bt,
            l0g, l0l, l0b, WL1, WG1, bL1, bG1):
    def body(sl_ref, sg_ref, fl_ref, fg_ref, dl_ref, dg_ref, gm_ref, bt_ref,
             l0g_ref, l0l_ref, l0b_ref, wl1_ref, wg1_ref, bl1_ref, bg1_ref,
             hL1p_ref, hG1p_ref, selfL1_ref, selfG1_ref, x1_ref):
        dl = lax.rsqrt(dl_ref[...] + 2.0)
        dg = lax.rsqrt(dg_ref[...] + 2.0)
        xL = _ln_relu(dl * sl_ref[...] + fl_ref[...], gm_ref[...], bt_ref[...])
        xG = _ln_relu(dg * sg_ref[...] + fg_ref[...], gm_ref[...], bt_ref[...])
        x1 = (jnp.dot(xG, l0g_ref[...], preferred_element_type=jnp.float32)
              + jnp.dot(xL, l0l_ref[...], preferred_element_type=jnp.float32)
              + l0b_ref[...])
        hl1 = jnp.dot(x1, wl1_ref[...], preferred_element_type=jnp.float32)
        hg1 = jnp.dot(x1, wg1_ref[...], preferred_element_type=jnp.float32)
        hL1p_ref[...] = dl * hl1
        hG1p_ref[...] = dg * hg1
        selfL1_ref[...] = 2.0 * dl * dl * hl1 + bl1_ref[...]
        selfG1_ref[...] = 2.0 * dg * dg * hg1 + bg1_ref[...]
        x1_ref[...] = x1

    return pl.pallas_call(
        body,
        grid=(_NP // _B,),
        in_specs=[_row, _row, _row, _row, _deg, _deg, _vec, _vec,
                  _mat, _mat, _vec, _mat, _mat, _vec, _vec],
        out_specs=[_row, _row, _row, _row, _row],
        out_shape=[jax.ShapeDtypeStruct((_NP, _H), jnp.float32)] * 5,
    )(sL, sG, selfL, selfG, degLb, degGb, gm, bt,
      l0g, l0l, l0b, WL1, WG1, bL1, bG1)


def _tc_post(sL, sG, selfL, selfG, degLb, degGb, x1, gm, bt,
             l1g, l1l, l1b, finW, finb):
    def body(sl_ref, sg_ref, fl_ref, fg_ref, dl_ref, dg_ref, x1_ref,
             gm_ref, bt_ref, l1g_ref, l1l_ref, l1b_ref, fw_ref, fb_ref,
             xf_ref, x2_ref):
        dl = lax.rsqrt(dl_ref[...] + 2.0)
        dg = lax.rsqrt(dg_ref[...] + 2.0)
        x1 = x1_ref[...]
        xL = _ln_relu(dl * sl_ref[...] + fl_ref[...],
                      gm_ref[...], bt_ref[...]) + x1
        xG = _ln_relu(dg * sg_ref[...] + fg_ref[...],
                      gm_ref[...], bt_ref[...]) + x1
        x2 = (jnp.dot(xG, l1g_ref[...], preferred_element_type=jnp.float32)
              + jnp.dot(xL, l1l_ref[...], preferred_element_type=jnp.float32)
              + l1b_ref[...])
        xf_ref[...] = (jnp.dot(x2, fw_ref[...],
                               preferred_element_type=jnp.float32)
                       + fb_ref[...])
        x2_ref[...] = x2

    return pl.pallas_call(
        body,
        grid=(_NP // _B,),
        in_specs=[_row, _row, _row, _row, _deg, _deg, _row, _vec, _vec,
                  _mat, _mat, _vec, _mat, _vec],
        out_specs=[_row, _row],
        out_shape=[jax.ShapeDtypeStruct((_NP, _H), jnp.float32)] * 2,
    )(sL, sG, selfL, selfG, degLb, degGb, x1, gm, bt,
      l1g, l1l, l1b, finW, finb)


def kernel(x, WL0, bL0, WG0, bG0, lin0W, lin0b, WL1, bL1, WG1, bG1,
           lin1W, lin1b, gamma, beta, finW, finb,
           G_edge_index, L_edge_index):
    degLr, degGr = _sc_degree(L_edge_index[1].reshape(_E // _CH, _CH),
                              G_edge_index[1].reshape(_E // _CH, _CH))
    degLb = degLr.reshape(_NP, 1)
    degGb = degGr.reshape(_NP, 1)
    xp = jnp.pad(x, ((0, _NP - _N), (0, 0)))

    bL0r = bL0.reshape(1, _H)
    bG0r = bG0.reshape(1, _H)
    bL1r = bL1.reshape(1, _H)
    bG1r = bG1.reshape(1, _H)
    gm = gamma.reshape(1, _H)
    bt = beta.reshape(1, _H)
    l0b = lin0b.reshape(1, _H)
    l1b = lin1b.reshape(1, _H)
    fb = finb.reshape(1, _H)

    hLp, hGp, selfL, selfG = _tc_pre(xp, WL0, WG0, bL0r, bG0r, degLb, degGb)
    srcL, dstL = L_edge_index[0], L_edge_index[1]
    srcG, dstG = G_edge_index[0], G_edge_index[1]
    sL, sG = _sc_gather_scatter(hLp, hGp, srcL, dstL, srcG, dstG)
    hL1p, hG1p, selfL1, selfG1, x1 = _tc_mid(
        sL, sG, selfL, selfG, degLb, degGb, gm, bt,
        lin0W[:_H], lin0W[_H:], l0b, WL1, WG1, bL1r, bG1r)
    sL1, sG1 = _sc_gather_scatter(hL1p, hG1p, srcL, dstL, srcG, dstG)
    xf, x2 = _tc_post(sL1, sG1, selfL1, selfG1, degLb, degGb, x1,
                      gm, bt, lin1W[:_H], lin1W[_H:], l1b, finW, fb)
    return (xf[:_N], x2[:_N])
